# baseline (speedup 1.0000x reference)
import functools

import jax
import jax.numpy as jnp
from jax import lax
from jax.experimental import pallas as pl
from jax.experimental.pallas import tpu as pltpu
from jax.experimental.pallas import tpu_sc as plsc

N_NODES = 10000
N_PAD = 10240
D = 128
NC, NS = 2, 16
NW = NC * NS
EBLK = 64
ROWS_PER_TILE = N_PAD // NS

_f32 = jnp.float32


def _att_tail(i, h, as_ref, ad_ref, asv_ref, adv_ref, mas_ref, mad_ref):
    asv = (h * as_ref[...]).sum(axis=1)
    adv = (h * ad_ref[...]).sum(axis=1)
    asv_ref[...] = asv.reshape(1, 1, D)
    adv_ref[...] = adv.reshape(1, 1, D)

    @pl.when(i == 0)
    def _():
        mas_ref[...] = jnp.full((1, 1), -1e30, _f32)
        mad_ref[...] = jnp.full((1, 1), -1e30, _f32)

    mas_ref[...] = jnp.maximum(mas_ref[...], jnp.max(asv).reshape(1, 1))
    mad_ref[...] = jnp.maximum(mad_ref[...], jnp.max(adv).reshape(1, 1))


def _mm_att_body(x_ref, w_ref, as_ref, ad_ref, h_ref, asv_ref, adv_ref,
                 mas_ref, mad_ref):
    h = jnp.dot(x_ref[...], w_ref[...], preferred_element_type=_f32)
    h_ref[...] = h
    _att_tail(pl.program_id(0), h, as_ref, ad_ref, asv_ref, adv_ref,
              mas_ref, mad_ref)


def _mm_att(x_pad, W, a_s, a_d):
    nb = N_PAD // 128
    h, as2d, ad2d, mas, mad = pl.pallas_call(
        _mm_att_body,
        grid=(nb,),
        in_specs=[
            pl.BlockSpec((128, D), lambda i: (i, 0)),
            pl.BlockSpec((D, D), lambda i: (0, 0)),
            pl.BlockSpec((1, D), lambda i: (0, 0)),
            pl.BlockSpec((1, D), lambda i: (0, 0)),
        ],
        out_specs=[
            pl.BlockSpec((128, D), lambda i: (i, 0)),
            pl.BlockSpec((1, 1, 128), lambda i: (i, 0, 0)),
            pl.BlockSpec((1, 1, 128), lambda i: (i, 0, 0)),
            pl.BlockSpec((1, 1), lambda i: (0, 0)),
            pl.BlockSpec((1, 1), lambda i: (0, 0)),
        ],
        out_shape=[
            jax.ShapeDtypeStruct((N_PAD, D), _f32),
            jax.ShapeDtypeStruct((nb, 1, 128), _f32),
            jax.ShapeDtypeStruct((nb, 1, 128), _f32),
            jax.ShapeDtypeStruct((1, 1), _f32),
            jax.ShapeDtypeStruct((1, 1), _f32),
        ],
    )(x_pad, W, a_s.reshape(1, D), a_d.reshape(1, D))
    return h, as2d.reshape(-1), ad2d.reshape(-1), _cshift16(mas, mad)


def _cshift16(mas, mad):
    craw = mas[0, 0] + mad[0, 0]
    cshift = jnp.maximum(craw, 0.2 * craw)
    return jnp.full((16,), cshift, _f32)


def _comb_mm_body(a0_ref, a1_ref, d0_ref, d1_ref, b_ref, w_ref, as_ref,
                  ad_ref, h_ref, asv_ref, adv_ref, mas_ref, mad_ref):
    i = pl.program_id(0)
    den = d0_ref[...] + d1_ref[...] + 1e-16
    xb = (a0_ref[...] + a1_ref[...]) / den + b_ref[...]
    xb = jnp.maximum(xb, 0.0)
    gid = i * 128 + lax.broadcasted_iota(jnp.int32, (128, D), 0)
    xb = jnp.where(gid < N_NODES, xb, 0.0)
    h = jnp.dot(xb, w_ref[...], preferred_element_type=_f32)
    h_ref[...] = h
    _att_tail(i, h, as_ref, ad_ref, asv_ref, adv_ref, mas_ref, mad_ref)


def _comb_mm(acc, den, b, W, a_s, a_d):
    nb = N_PAD // 128
    h, as2d, ad2d, mas, mad = pl.pallas_call(
        _comb_mm_body,
        grid=(nb,),
        in_specs=[
            pl.BlockSpec((128, D), lambda i: (i, 0)),
            pl.BlockSpec((128, D), lambda i: (i, 0)),
            pl.BlockSpec((128, 1), lambda i: (i, 0)),
            pl.BlockSpec((128, 1), lambda i: (i, 0)),
            pl.BlockSpec((1, D), lambda i: (0, 0)),
            pl.BlockSpec((D, D), lambda i: (0, 0)),
            pl.BlockSpec((1, D), lambda i: (0, 0)),
            pl.BlockSpec((1, D), lambda i: (0, 0)),
        ],
        out_specs=[
            pl.BlockSpec((128, D), lambda i: (i, 0)),
            pl.BlockSpec((1, 1, 128), lambda i: (i, 0, 0)),
            pl.BlockSpec((1, 1, 128), lambda i: (i, 0, 0)),
            pl.BlockSpec((1, 1), lambda i: (0, 0)),
            pl.BlockSpec((1, 1), lambda i: (0, 0)),
        ],
        out_shape=[
            jax.ShapeDtypeStruct((N_PAD, D), _f32),
            jax.ShapeDtypeStruct((nb, 1, 128), _f32),
            jax.ShapeDtypeStruct((nb, 1, 128), _f32),
            jax.ShapeDtypeStruct((1, 1), _f32),
            jax.ShapeDtypeStruct((1, 1), _f32),
        ],
    )(acc[0], acc[1], den[0].reshape(N_PAD, 1), den[1].reshape(N_PAD, 1),
      b.reshape(1, D), W, a_s.reshape(1, D), a_d.reshape(1, D))
    return h, as2d.reshape(-1), ad2d.reshape(-1), _cshift16(mas, mad)


def _final_body(a0_ref, a1_ref, d0_ref, d1_ref, b_ref, o_ref):
    den = d0_ref[...] + d1_ref[...] + 1e-16
    xb = (a0_ref[...] + a1_ref[...]) / den + b_ref[...]
    o_ref[...] = jnp.maximum(xb, 0.0)


def _final(acc, den, b):
    nb = N_PAD // 128
    return pl.pallas_call(
        _final_body,
        grid=(nb,),
        in_specs=[
            pl.BlockSpec((128, D), lambda i: (i, 0)),
            pl.BlockSpec((128, D), lambda i: (i, 0)),
            pl.BlockSpec((128, 1), lambda i: (i, 0)),
            pl.BlockSpec((128, 1), lambda i: (i, 0)),
            pl.BlockSpec((1, D), lambda i: (0, 0)),
        ],
        out_specs=pl.BlockSpec((128, D), lambda i: (i, 0)),
        out_shape=jax.ShapeDtypeStruct((N_PAD, D), _f32),
    )(acc[0], acc[1], den[0].reshape(N_PAD, 1), den[1].reshape(N_PAD, 1),
      b.reshape(1, D))


def _sc_edge_pass(h, asv, adv, c16, src_p, dst_p, nblk):
    mesh = plsc.VectorSubcoreMesh(core_axis_name="c", subcore_axis_name="s",
                                  num_cores=NC, num_subcores=NS)

    @functools.partial(
        pl.kernel,
        out_type=[
            jax.ShapeDtypeStruct((NC, N_PAD, D), _f32),
            jax.ShapeDtypeStruct((NC, N_PAD), _f32),
        ],
        mesh=mesh,
        compiler_params=pltpu.CompilerParams(needs_layout_passes=False),
        scratch_types=[
            pltpu.VMEM((N_PAD,), _f32),
            pltpu.VMEM((N_PAD,), _f32),
            pltpu.VMEM((16,), _f32),
            pltpu.VMEM((4, EBLK), jnp.int32),
            pltpu.VMEM((4, EBLK), jnp.int32),
            pltpu.VMEM((2, EBLK, D), _f32),
            pltpu.VMEM((2, EBLK), _f32),
            pltpu.VMEM((ROWS_PER_TILE,), _f32),
            pltpu.VMEM_SHARED((N_PAD, D), _f32),
            pltpu.VMEM_SHARED((N_PAD,), _f32),
            pltpu.SemaphoreType.DMA((2,)),
            pltpu.SemaphoreType.DMA((4,)),
            pltpu.SemaphoreType.DMA((2,)),
            pltpu.SemaphoreType.DMA((2,)),
        ],
    )
    def body(h_hbm, as_hbm, ad_hbm, c_hbm, src_hbm, dst_hbm, acc_hbm,
             den_hbm, as_v, ad_v, c_v, src_v, dst_v, rows_v, w_v, dz_v,
             acc_sh, den_sh, sem_g, sem_i, sem_r, sem_w):
        c = lax.axis_index("c")
        s = lax.axis_index("s")

        pltpu.sync_copy(as_hbm, as_v)
        pltpu.sync_copy(ad_hbm, ad_v)
        pltpu.sync_copy(c_hbm, c_v)
        for slot in range(2):
            jj = min(slot, nblk - 1)
            pltpu.async_copy(src_hbm.at[c, s, jj], src_v.at[slot],
                             sem_i.at[slot])
            pltpu.async_copy(dst_hbm.at[c, s, jj], dst_v.at[slot],
                             sem_i.at[slot])

        zero16 = jnp.zeros((16,), _f32)

        def zrow(i, _):
            for k in range(D // 16):
                rows_v[0, i, pl.ds(k * 16, 16)] = zero16
            return 0

        lax.fori_loop(0, EBLK, zrow, 0)

        def zden(i, _):
            dz_v[pl.ds(i * 16, 16)] = zero16
            return 0

        lax.fori_loop(0, ROWS_PER_TILE // 16, zden, 0)

        base = s * ROWS_PER_TILE
        for r in range(ROWS_PER_TILE // EBLK):
            pltpu.sync_copy(rows_v.at[0],
                            acc_sh.at[pl.ds(base + r * EBLK, EBLK)])
        pltpu.sync_copy(dz_v, den_sh.at[pl.ds(base, ROWS_PER_TILE)])
        plsc.subcore_barrier()

        cshift = c_v[...]

        def blk(j, _):
            rs = lax.rem(j, 2)
            qs = lax.rem(j, 4)
            pltpu.make_async_copy(src_hbm.at[c, s, 0], src_v.at[qs],
                                  sem_i.at[qs]).wait()
            pltpu.make_async_copy(dst_hbm.at[c, s, 0], dst_v.at[qs],
                                  sem_i.at[qs]).wait()

            @pl.when(j >= 2)
            def _():
                pltpu.make_async_copy(
                    rows_v.at[rs], acc_sh.at[dst_v.at[qs]],
                    sem_r.at[rs]).wait()
                pltpu.make_async_copy(
                    w_v.at[rs], den_sh.at[dst_v.at[qs]],
                    sem_w.at[rs]).wait()

            cp = pltpu.async_copy(h_hbm.at[src_v.at[qs]],
                                  rows_v.at[rs], sem_g.at[rs])
            for k in range(EBLK // 16):
                idx_s = src_v[qs, pl.ds(k * 16, 16)]
                idx_d = dst_v[qs, pl.ds(k * 16, 16)]
                raw = (plsc.load_gather(as_v, [idx_s])
                       + plsc.load_gather(ad_v, [idx_d]))
                e = jnp.maximum(raw, 0.2 * raw)
                w_v[rs, pl.ds(k * 16, 16)] = jnp.exp(e - cshift)
            cp.wait()

            def scale(g, _):
                w16 = w_v[rs, pl.ds(g * 16, 16)]
                for lane in range(16):
                    wv = jnp.full((16,), w16[lane], _f32)
                    ei = g * 16 + lane
                    for k in range(D // 16):
                        rows_v[rs, ei, pl.ds(k * 16, 16)] = (
                            rows_v[rs, ei, pl.ds(k * 16, 16)] * wv)
                return 0

            lax.fori_loop(0, EBLK // 16, scale, 0)
            pltpu.async_copy(rows_v.at[rs], acc_sh.at[dst_v.at[qs]],
                             sem_r.at[rs], add=True)
            pltpu.async_copy(w_v.at[rs], den_sh.at[dst_v.at[qs]],
                             sem_w.at[rs], add=True)
            jn = jnp.minimum(j + 2, nblk - 1)
            qn = lax.rem(j + 2, 4)
            pltpu.async_copy(src_hbm.at[c, s, jn], src_v.at[qn],
                             sem_i.at[qn])
            pltpu.async_copy(dst_hbm.at[c, s, jn], dst_v.at[qn],
                             sem_i.at[qn])
            return 0

        lax.fori_loop(0, nblk, blk, 0)
        for tail in range(2):
            qs = (nblk + tail) % 4
            pltpu.make_async_copy(src_hbm.at[c, s, 0], src_v.at[qs],
                                  sem_i.at[qs]).wait()
            pltpu.make_async_copy(dst_hbm.at[c, s, 0], dst_v.at[qs],
                                  sem_i.at[qs]).wait()
        for rs in range(2):
            pltpu.make_async_copy(rows_v.at[rs],
                                  acc_sh.at[dst_v.at[rs]],
                                  sem_r.at[rs]).wait()
            pltpu.make_async_copy(w_v.at[rs],
                                  den_sh.at[dst_v.at[rs]],
                                  sem_w.at[rs]).wait()
        plsc.subcore_barrier()

        for r in range(ROWS_PER_TILE // EBLK):
            sl = pl.ds(base + r * EBLK, EBLK)
            pltpu.sync_copy(acc_sh.at[sl], acc_hbm.at[c, sl])
        pltpu.sync_copy(den_sh.at[pl.ds(base, ROWS_PER_TILE)],
                        den_hbm.at[c, pl.ds(base, ROWS_PER_TILE)])

    return body(h, asv, adv, c16, src_p, dst_p)


def kernel(x, edge_index, W1, a_src1, a_dst1, b1, W2, a_src2, a_dst2, b2):
    n = x.shape[0]
    loops = jnp.arange(n, dtype=jnp.int32)
    src = jnp.concatenate([edge_index[0].astype(jnp.int32), loops])
    dst = jnp.concatenate([edge_index[1].astype(jnp.int32), loops])
    e_total = src.shape[0]
    nblk = -(-e_total // (NW * EBLK))
    e_pad = NW * nblk * EBLK
    sent = jnp.int32(n)
    src_p = jnp.full((e_pad,), sent, jnp.int32).at[:e_total].set(src)
    dst_p = jnp.full((e_pad,), sent, jnp.int32).at[:e_total].set(dst)
    src_p = src_p.reshape(NC, NS, nblk, EBLK)
    dst_p = dst_p.reshape(NC, NS, nblk, EBLK)

    x_pad = jnp.pad(x, ((0, N_PAD - n), (0, 0)))
    h1, as1v, ad1v, c1 = _mm_att(x_pad, W1, a_src1, a_dst1)
    acc1, den1 = _sc_edge_pass(h1, as1v, ad1v, c1, src_p, dst_p, nblk)
    h2, as2v, ad2v, c2 = _comb_mm(acc1, den1, b1, W2, a_src2, a_dst2)
    acc2, den2 = _sc_edge_pass(h2, as2v, ad2v, c2, src_p, dst_p, nblk)
    out = _final(acc2, den2, b2)
    return out[:n]


# speedup vs baseline: 1.0414x; 1.0414x over previous
import functools

import jax
import jax.numpy as jnp
from jax import lax
from jax.experimental import pallas as pl
from jax.experimental.pallas import tpu as pltpu
from jax.experimental.pallas import tpu_sc as plsc

N_NODES = 10000
N_PAD = 10240
D = 128
NC, NS = 2, 16
NW = NC * NS
EBLK = 128
CH = 9
ROWS_PER_TILE = N_PAD // NS

_f32 = jnp.float32


def _att_tail(i, h, as_ref, ad_ref, asv_ref, adv_ref, mas_ref, mad_ref):
    asv = (h * as_ref[...]).sum(axis=1)
    adv = (h * ad_ref[...]).sum(axis=1)
    asv_ref[...] = asv.reshape(1, 1, D)
    adv_ref[...] = adv.reshape(1, 1, D)

    @pl.when(i == 0)
    def _():
        mas_ref[...] = jnp.full((1, 1), -1e30, _f32)
        mad_ref[...] = jnp.full((1, 1), -1e30, _f32)

    mas_ref[...] = jnp.maximum(mas_ref[...], jnp.max(asv).reshape(1, 1))
    mad_ref[...] = jnp.maximum(mad_ref[...], jnp.max(adv).reshape(1, 1))


def _mm_att_body(x_ref, w_ref, as_ref, ad_ref, h_ref, asv_ref, adv_ref,
                 mas_ref, mad_ref):
    h = jnp.dot(x_ref[...], w_ref[...], preferred_element_type=_f32)
    h_ref[...] = h
    _att_tail(pl.program_id(0), h, as_ref, ad_ref, asv_ref, adv_ref,
              mas_ref, mad_ref)


def _mm_att(x_pad, W, a_s, a_d):
    nb = N_PAD // 128
    h, as2d, ad2d, mas, mad = pl.pallas_call(
        _mm_att_body,
        grid=(nb,),
        in_specs=[
            pl.BlockSpec((128, D), lambda i: (i, 0)),
            pl.BlockSpec((D, D), lambda i: (0, 0)),
            pl.BlockSpec((1, D), lambda i: (0, 0)),
            pl.BlockSpec((1, D), lambda i: (0, 0)),
        ],
        out_specs=[
            pl.BlockSpec((128, D), lambda i: (i, 0)),
            pl.BlockSpec((1, 1, 128), lambda i: (i, 0, 0)),
            pl.BlockSpec((1, 1, 128), lambda i: (i, 0, 0)),
            pl.BlockSpec((1, 1), lambda i: (0, 0)),
            pl.BlockSpec((1, 1), lambda i: (0, 0)),
        ],
        out_shape=[
            jax.ShapeDtypeStruct((N_PAD, D), _f32),
            jax.ShapeDtypeStruct((nb, 1, 128), _f32),
            jax.ShapeDtypeStruct((nb, 1, 128), _f32),
            jax.ShapeDtypeStruct((1, 1), _f32),
            jax.ShapeDtypeStruct((1, 1), _f32),
        ],
    )(x_pad, W, a_s.reshape(1, D), a_d.reshape(1, D))
    return h, as2d.reshape(-1), ad2d.reshape(-1), _cshift16(mas, mad)


def _cshift16(mas, mad):
    craw = mas[0, 0] + mad[0, 0]
    cshift = jnp.maximum(craw, 0.2 * craw)
    return jnp.full((16,), cshift, _f32)


def _comb_mm_body(a0_ref, a1_ref, d0_ref, d1_ref, b_ref, w_ref, as_ref,
                  ad_ref, h_ref, asv_ref, adv_ref, mas_ref, mad_ref):
    i = pl.program_id(0)
    den = d0_ref[...] + d1_ref[...] + 1e-16
    xb = (a0_ref[...] + a1_ref[...]) / den + b_ref[...]
    xb = jnp.maximum(xb, 0.0)
    gid = i * 128 + lax.broadcasted_iota(jnp.int32, (128, D), 0)
    xb = jnp.where(gid < N_NODES, xb, 0.0)
    h = jnp.dot(xb, w_ref[...], preferred_element_type=_f32)
    h_ref[...] = h
    _att_tail(i, h, as_ref, ad_ref, asv_ref, adv_ref, mas_ref, mad_ref)


def _comb_mm(acc, den, b, W, a_s, a_d):
    nb = N_PAD // 128
    h, as2d, ad2d, mas, mad = pl.pallas_call(
        _comb_mm_body,
        grid=(nb,),
        in_specs=[
            pl.BlockSpec((128, D), lambda i: (i, 0)),
            pl.BlockSpec((128, D), lambda i: (i, 0)),
            pl.BlockSpec((128, 1), lambda i: (i, 0)),
            pl.BlockSpec((128, 1), lambda i: (i, 0)),
            pl.BlockSpec((1, D), lambda i: (0, 0)),
            pl.BlockSpec((D, D), lambda i: (0, 0)),
            pl.BlockSpec((1, D), lambda i: (0, 0)),
            pl.BlockSpec((1, D), lambda i: (0, 0)),
        ],
        out_specs=[
            pl.BlockSpec((128, D), lambda i: (i, 0)),
            pl.BlockSpec((1, 1, 128), lambda i: (i, 0, 0)),
            pl.BlockSpec((1, 1, 128), lambda i: (i, 0, 0)),
            pl.BlockSpec((1, 1), lambda i: (0, 0)),
            pl.BlockSpec((1, 1), lambda i: (0, 0)),
        ],
        out_shape=[
            jax.ShapeDtypeStruct((N_PAD, D), _f32),
            jax.ShapeDtypeStruct((nb, 1, 128), _f32),
            jax.ShapeDtypeStruct((nb, 1, 128), _f32),
            jax.ShapeDtypeStruct((1, 1), _f32),
            jax.ShapeDtypeStruct((1, 1), _f32),
        ],
    )(acc[0], acc[1], den[0].reshape(N_PAD, 1), den[1].reshape(N_PAD, 1),
      b.reshape(1, D), W, a_s.reshape(1, D), a_d.reshape(1, D))
    return h, as2d.reshape(-1), ad2d.reshape(-1), _cshift16(mas, mad)


def _final_body(a0_ref, a1_ref, d0_ref, d1_ref, b_ref, o_ref):
    den = d0_ref[...] + d1_ref[...] + 1e-16
    xb = (a0_ref[...] + a1_ref[...]) / den + b_ref[...]
    o_ref[...] = jnp.maximum(xb, 0.0)


def _final(acc, den, b):
    nb = N_PAD // 128
    return pl.pallas_call(
        _final_body,
        grid=(nb,),
        in_specs=[
            pl.BlockSpec((128, D), lambda i: (i, 0)),
            pl.BlockSpec((128, D), lambda i: (i, 0)),
            pl.BlockSpec((128, 1), lambda i: (i, 0)),
            pl.BlockSpec((128, 1), lambda i: (i, 0)),
            pl.BlockSpec((1, D), lambda i: (0, 0)),
        ],
        out_specs=pl.BlockSpec((128, D), lambda i: (i, 0)),
        out_shape=jax.ShapeDtypeStruct((N_PAD, D), _f32),
    )(acc[0], acc[1], den[0].reshape(N_PAD, 1), den[1].reshape(N_PAD, 1),
      b.reshape(1, D))


def _sc_edge_pass(h, asv, adv, c16, src_p, dst_p, nch):
    nblk = nch * CH
    mesh = plsc.VectorSubcoreMesh(core_axis_name="c", subcore_axis_name="s",
                                  num_cores=NC, num_subcores=NS)

    @functools.partial(
        pl.kernel,
        out_type=[
            jax.ShapeDtypeStruct((NC, N_PAD, D), _f32),
            jax.ShapeDtypeStruct((NC, N_PAD), _f32),
        ],
        mesh=mesh,
        compiler_params=pltpu.CompilerParams(needs_layout_passes=False),
        scratch_types=[
            pltpu.VMEM((16,), _f32),
            pltpu.VMEM((3, CH, EBLK), jnp.int32),
            pltpu.VMEM((3, CH, EBLK), jnp.int32),
            pltpu.VMEM((2, EBLK, D), _f32),
            pltpu.VMEM((2, EBLK), _f32),
            pltpu.VMEM((2, EBLK), _f32),
            pltpu.VMEM((2, EBLK), _f32),
            pltpu.VMEM((ROWS_PER_TILE,), _f32),
            pltpu.VMEM_SHARED((N_PAD, D), _f32),
            pltpu.VMEM_SHARED((N_PAD,), _f32),
            pltpu.SemaphoreType.DMA((2,)),
            pltpu.SemaphoreType.DMA((3,)),
            pltpu.SemaphoreType.DMA((2,)),
            pltpu.SemaphoreType.DMA((2,)),
            pltpu.SemaphoreType.DMA((2,)),
        ],
    )
    def body(h_hbm, as_hbm, ad_hbm, c_hbm, src_hbm, dst_hbm, acc_hbm,
             den_hbm, c_v, src_st, dst_st, rows_v, w_v, asg_v, adg_v,
             dz_v, acc_sh, den_sh, sem_g, sem_ix, sem_r, sem_w, sem_a):
        c = lax.axis_index("c")
        s = lax.axis_index("s")

        pltpu.sync_copy(c_hbm, c_v)
        for k in range(min(2, nch)):
            pltpu.async_copy(src_hbm.at[c, s, k], src_st.at[k],
                             sem_ix.at[k])
            pltpu.async_copy(dst_hbm.at[c, s, k], dst_st.at[k],
                             sem_ix.at[k])

        zero16 = jnp.zeros((16,), _f32)

        def zrow(i, _):
            for k in range(D // 16):
                rows_v[0, i, pl.ds(k * 16, 16)] = zero16
            return 0

        lax.fori_loop(0, EBLK, zrow, 0)

        def zden(i, _):
            dz_v[pl.ds(i * 16, 16)] = zero16
            return 0

        lax.fori_loop(0, ROWS_PER_TILE // 16, zden, 0)

        base = s * ROWS_PER_TILE
        for r in range(ROWS_PER_TILE // EBLK):
            pltpu.sync_copy(rows_v.at[0],
                            acc_sh.at[pl.ds(base + r * EBLK, EBLK)])
        pltpu.sync_copy(dz_v, den_sh.at[pl.ds(base, ROWS_PER_TILE)])
        plsc.subcore_barrier()

        cshift = c_v[...]

        def blk(j, _):
            rs = lax.rem(j, 2)
            chunk = lax.div(j, CH)
            within = lax.rem(j, CH)
            cs = lax.rem(chunk, 3)

            @pl.when(within == 0)
            def _():
                pltpu.make_async_copy(src_hbm.at[c, s, 0], src_st.at[cs],
                                      sem_ix.at[cs]).wait()
                pltpu.make_async_copy(dst_hbm.at[c, s, 0], dst_st.at[cs],
                                      sem_ix.at[cs]).wait()

            @pl.when(j >= 2)
            def _():
                pltpu.make_async_copy(
                    rows_v.at[rs], acc_sh.at[dst_st.at[cs, 0]],
                    sem_r.at[rs]).wait()
                pltpu.make_async_copy(
                    w_v.at[rs], den_sh.at[dst_st.at[cs, 0]],
                    sem_w.at[rs]).wait()

            @pl.when(jnp.logical_and(within == 2, chunk + 2 < nch))
            def _():
                cn = lax.rem(chunk + 2, 3)
                pltpu.async_copy(src_hbm.at[c, s, chunk + 2],
                                 src_st.at[cn], sem_ix.at[cn])
                pltpu.async_copy(dst_hbm.at[c, s, chunk + 2],
                                 dst_st.at[cn], sem_ix.at[cn])

            srow = src_st.at[cs, within]
            drow = dst_st.at[cs, within]
            cp_h = pltpu.async_copy(h_hbm.at[srow], rows_v.at[rs],
                                    sem_g.at[rs])
            cp_a = pltpu.async_copy(as_hbm.at[srow], asg_v.at[rs],
                                    sem_a.at[rs])
            cp_b = pltpu.async_copy(ad_hbm.at[drow], adg_v.at[rs],
                                    sem_a.at[rs])
            cp_a.wait()
            cp_b.wait()
            for k in range(EBLK // 16):
                raw = (asg_v[rs, pl.ds(k * 16, 16)]
                       + adg_v[rs, pl.ds(k * 16, 16)])
                e = jnp.maximum(raw, 0.2 * raw)
                w_v[rs, pl.ds(k * 16, 16)] = jnp.exp(e - cshift)
            cp_h.wait()

            def scale(g, _):
                w16 = w_v[rs, pl.ds(g * 16, 16)]
                for lane in range(16):
                    wv = jnp.full((16,), w16[lane], _f32)
                    ei = g * 16 + lane
                    for k in range(D // 16):
                        rows_v[rs, ei, pl.ds(k * 16, 16)] = (
                            rows_v[rs, ei, pl.ds(k * 16, 16)] * wv)
                return 0

            lax.fori_loop(0, EBLK // 16, scale, 0)
            pltpu.async_copy(rows_v.at[rs], acc_sh.at[drow],
                             sem_r.at[rs], add=True)
            pltpu.async_copy(w_v.at[rs], den_sh.at[drow],
                             sem_w.at[rs], add=True)
            return 0

        lax.fori_loop(0, nblk, blk, 0)
        for rs in range(2):
            pltpu.make_async_copy(rows_v.at[rs],
                                  acc_sh.at[dst_st.at[0, 0]],
                                  sem_r.at[rs]).wait()
            pltpu.make_async_copy(w_v.at[rs],
                                  den_sh.at[dst_st.at[0, 0]],
                                  sem_w.at[rs]).wait()
        plsc.subcore_barrier()

        for r in range(ROWS_PER_TILE // EBLK):
            sl = pl.ds(base + r * EBLK, EBLK)
            pltpu.sync_copy(acc_sh.at[sl], acc_hbm.at[c, sl])
        pltpu.sync_copy(den_sh.at[pl.ds(base, ROWS_PER_TILE)],
                        den_hbm.at[c, pl.ds(base, ROWS_PER_TILE)])

    return body(h, asv, adv, c16, src_p, dst_p)


def kernel(x, edge_index, W1, a_src1, a_dst1, b1, W2, a_src2, a_dst2, b2):
    n = x.shape[0]
    loops = jnp.arange(n, dtype=jnp.int32)
    src = jnp.concatenate([edge_index[0].astype(jnp.int32), loops])
    dst = jnp.concatenate([edge_index[1].astype(jnp.int32), loops])
    e_total = src.shape[0]
    nch = -(-e_total // (NW * CH * EBLK))
    e_pad = NW * nch * CH * EBLK
    sent = jnp.int32(n)
    src_p = jnp.full((e_pad,), sent, jnp.int32).at[:e_total].set(src)
    dst_p = jnp.full((e_pad,), sent, jnp.int32).at[:e_total].set(dst)
    src_p = src_p.reshape(NC, NS, nch, CH, EBLK)
    dst_p = dst_p.reshape(NC, NS, nch, CH, EBLK)

    x_pad = jnp.pad(x, ((0, N_PAD - n), (0, 0)))
    h1, as1v, ad1v, c1 = _mm_att(x_pad, W1, a_src1, a_dst1)
    acc1, den1 = _sc_edge_pass(h1, as1v, ad1v, c1, src_p, dst_p, nch)
    h2, as2v, ad2v, c2 = _comb_mm(acc1, den1, b1, W2, a_src2, a_dst2)
    acc2, den2 = _sc_edge_pass(h2, as2v, ad2v, c2, src_p, dst_p, nch)
    out = _final(acc2, den2, b2)
    return out[:n]


# speedup vs baseline: 1.6594x; 1.5935x over previous
import functools

import jax
import jax.numpy as jnp
from jax import lax
from jax.experimental import pallas as pl
from jax.experimental.pallas import tpu as pltpu
from jax.experimental.pallas import tpu_sc as plsc

N_NODES = 10000
N_PAD = 10240
D = 128
NC, NS = 2, 16
NW = NC * NS
EBLK = 128
CH = 9
ROWS_PER_TILE = N_PAD // NS

_f32 = jnp.float32


def _att_tail(i, h, as_ref, ad_ref, asv_ref, adv_ref, mas_ref, mad_ref):
    asv = (h * as_ref[...]).sum(axis=1)
    adv = (h * ad_ref[...]).sum(axis=1)
    asv_ref[...] = asv.reshape(1, 1, D)
    adv_ref[...] = adv.reshape(1, 1, D)

    @pl.when(i == 0)
    def _():
        mas_ref[...] = jnp.full((1, 1), -1e30, _f32)
        mad_ref[...] = jnp.full((1, 1), -1e30, _f32)

    mas_ref[...] = jnp.maximum(mas_ref[...], jnp.max(asv).reshape(1, 1))
    mad_ref[...] = jnp.maximum(mad_ref[...], jnp.max(adv).reshape(1, 1))


def _mm_att_body(x_ref, w_ref, as_ref, ad_ref, h_ref, asv_ref, adv_ref,
                 mas_ref, mad_ref):
    h = jnp.dot(x_ref[...], w_ref[...], preferred_element_type=_f32)
    h_ref[...] = h
    _att_tail(pl.program_id(0), h, as_ref, ad_ref, asv_ref, adv_ref,
              mas_ref, mad_ref)


def _mm_att(x_pad, W, a_s, a_d):
    nb = N_PAD // 128
    h, as2d, ad2d, mas, mad = pl.pallas_call(
        _mm_att_body,
        grid=(nb,),
        in_specs=[
            pl.BlockSpec((128, D), lambda i: (i, 0)),
            pl.BlockSpec((D, D), lambda i: (0, 0)),
            pl.BlockSpec((1, D), lambda i: (0, 0)),
            pl.BlockSpec((1, D), lambda i: (0, 0)),
        ],
        out_specs=[
            pl.BlockSpec((128, D), lambda i: (i, 0)),
            pl.BlockSpec((1, 1, 128), lambda i: (i, 0, 0)),
            pl.BlockSpec((1, 1, 128), lambda i: (i, 0, 0)),
            pl.BlockSpec((1, 1), lambda i: (0, 0)),
            pl.BlockSpec((1, 1), lambda i: (0, 0)),
        ],
        out_shape=[
            jax.ShapeDtypeStruct((N_PAD, D), _f32),
            jax.ShapeDtypeStruct((nb, 1, 128), _f32),
            jax.ShapeDtypeStruct((nb, 1, 128), _f32),
            jax.ShapeDtypeStruct((1, 1), _f32),
            jax.ShapeDtypeStruct((1, 1), _f32),
        ],
    )(x_pad, W, a_s.reshape(1, D), a_d.reshape(1, D))
    return h, as2d.reshape(-1), ad2d.reshape(-1), _cshift16(mas, mad)


def _cshift16(mas, mad):
    craw = mas[0, 0] + mad[0, 0]
    cshift = jnp.maximum(craw, 0.2 * craw)
    return jnp.full((16,), cshift, _f32)


def _comb_mm_body(a0_ref, a1_ref, d0_ref, d1_ref, b_ref, w_ref, as_ref,
                  ad_ref, h_ref, asv_ref, adv_ref, mas_ref, mad_ref):
    i = pl.program_id(0)
    den = d0_ref[...] + d1_ref[...] + 1e-16
    xb = (a0_ref[...] + a1_ref[...]) / den + b_ref[...]
    xb = jnp.maximum(xb, 0.0)
    gid = i * 128 + lax.broadcasted_iota(jnp.int32, (128, D), 0)
    xb = jnp.where(gid < N_NODES, xb, 0.0)
    h = jnp.dot(xb, w_ref[...], preferred_element_type=_f32)
    h_ref[...] = h
    _att_tail(i, h, as_ref, ad_ref, asv_ref, adv_ref, mas_ref, mad_ref)


def _comb_mm(acc, den, b, W, a_s, a_d):
    nb = N_PAD // 128
    h, as2d, ad2d, mas, mad = pl.pallas_call(
        _comb_mm_body,
        grid=(nb,),
        in_specs=[
            pl.BlockSpec((128, D), lambda i: (i, 0)),
            pl.BlockSpec((128, D), lambda i: (i, 0)),
            pl.BlockSpec((128, 1), lambda i: (i, 0)),
            pl.BlockSpec((128, 1), lambda i: (i, 0)),
            pl.BlockSpec((1, D), lambda i: (0, 0)),
            pl.BlockSpec((D, D), lambda i: (0, 0)),
            pl.BlockSpec((1, D), lambda i: (0, 0)),
            pl.BlockSpec((1, D), lambda i: (0, 0)),
        ],
        out_specs=[
            pl.BlockSpec((128, D), lambda i: (i, 0)),
            pl.BlockSpec((1, 1, 128), lambda i: (i, 0, 0)),
            pl.BlockSpec((1, 1, 128), lambda i: (i, 0, 0)),
            pl.BlockSpec((1, 1), lambda i: (0, 0)),
            pl.BlockSpec((1, 1), lambda i: (0, 0)),
        ],
        out_shape=[
            jax.ShapeDtypeStruct((N_PAD, D), _f32),
            jax.ShapeDtypeStruct((nb, 1, 128), _f32),
            jax.ShapeDtypeStruct((nb, 1, 128), _f32),
            jax.ShapeDtypeStruct((1, 1), _f32),
            jax.ShapeDtypeStruct((1, 1), _f32),
        ],
    )(acc[0], acc[1], den[0].reshape(N_PAD, 1), den[1].reshape(N_PAD, 1),
      b.reshape(1, D), W, a_s.reshape(1, D), a_d.reshape(1, D))
    return h, as2d.reshape(-1), ad2d.reshape(-1), _cshift16(mas, mad)


def _final_body(a0_ref, a1_ref, d0_ref, d1_ref, b_ref, o_ref):
    den = d0_ref[...] + d1_ref[...] + 1e-16
    xb = (a0_ref[...] + a1_ref[...]) / den + b_ref[...]
    o_ref[...] = jnp.maximum(xb, 0.0)


def _final(acc, den, b):
    nb = N_PAD // 128
    return pl.pallas_call(
        _final_body,
        grid=(nb,),
        in_specs=[
            pl.BlockSpec((128, D), lambda i: (i, 0)),
            pl.BlockSpec((128, D), lambda i: (i, 0)),
            pl.BlockSpec((128, 1), lambda i: (i, 0)),
            pl.BlockSpec((128, 1), lambda i: (i, 0)),
            pl.BlockSpec((1, D), lambda i: (0, 0)),
        ],
        out_specs=pl.BlockSpec((128, D), lambda i: (i, 0)),
        out_shape=jax.ShapeDtypeStruct((N_PAD, D), _f32),
    )(acc[0], acc[1], den[0].reshape(N_PAD, 1), den[1].reshape(N_PAD, 1),
      b.reshape(1, D))


def _sc_edge_pass(h, asv, adv, c16, src_p, dst_p, nblk):
    mesh = plsc.VectorSubcoreMesh(core_axis_name="c", subcore_axis_name="s",
                                  num_cores=NC, num_subcores=NS)

    @functools.partial(
        pl.kernel,
        out_type=[
            jax.ShapeDtypeStruct((NC, N_PAD, D), _f32),
            jax.ShapeDtypeStruct((NC, N_PAD), _f32),
        ],
        mesh=mesh,
        compiler_params=pltpu.CompilerParams(needs_layout_passes=False),
        scratch_types=[
            pltpu.VMEM((N_PAD,), _f32),
            pltpu.VMEM((N_PAD,), _f32),
            pltpu.VMEM((16,), _f32),
            pltpu.VMEM((2, EBLK), jnp.int32),
            pltpu.VMEM((2, EBLK), jnp.int32),
            pltpu.VMEM((EBLK, D), _f32),
            pltpu.VMEM((EBLK,), _f32),
            pltpu.VMEM((ROWS_PER_TILE,), _f32),
            pltpu.VMEM_SHARED((N_PAD, D), _f32),
            pltpu.VMEM_SHARED((N_PAD,), _f32),
            pltpu.SemaphoreType.DMA,
            pltpu.SemaphoreType.DMA((2,)),
        ],
    )
    def body(h_hbm, as_hbm, ad_hbm, c_hbm, src_hbm, dst_hbm, acc_hbm,
             den_hbm, as_v, ad_v, c_v, src_v, dst_v, rows_v, w_v, dz_v,
             acc_sh, den_sh, sem, sem_i):
        c = lax.axis_index("c")
        s = lax.axis_index("s")

        pltpu.sync_copy(as_hbm, as_v)
        pltpu.sync_copy(ad_hbm, ad_v)
        pltpu.sync_copy(c_hbm, c_v)
        for slot in range(2):
            jj = min(slot, nblk - 1)
            pltpu.async_copy(src_hbm.at[c, s, jj], src_v.at[slot],
                             sem_i.at[slot])
            pltpu.async_copy(dst_hbm.at[c, s, jj], dst_v.at[slot],
                             sem_i.at[slot])

        zero16 = jnp.zeros((16,), _f32)

        def zrow(i, _):
            for k in range(D // 16):
                rows_v[i, pl.ds(k * 16, 16)] = zero16
            return 0

        lax.fori_loop(0, EBLK, zrow, 0)

        def zden(i, _):
            dz_v[pl.ds(i * 16, 16)] = zero16
            return 0

        lax.fori_loop(0, ROWS_PER_TILE // 16, zden, 0)

        base = s * ROWS_PER_TILE
        for r in range(ROWS_PER_TILE // EBLK):
            pltpu.sync_copy(rows_v, acc_sh.at[pl.ds(base + r * EBLK, EBLK)])
        pltpu.sync_copy(dz_v, den_sh.at[pl.ds(base, ROWS_PER_TILE)])
        plsc.subcore_barrier()

        cshift = c_v[...]

        def blk(j, _):
            slot = lax.rem(j, 2)
            pltpu.make_async_copy(src_hbm.at[c, s, 0], src_v.at[slot],
                                  sem_i.at[slot]).wait()
            pltpu.make_async_copy(dst_hbm.at[c, s, 0], dst_v.at[slot],
                                  sem_i.at[slot]).wait()
            cp = pltpu.async_copy(h_hbm.at[src_v.at[slot]], rows_v, sem)
            for k in range(EBLK // 16):
                idx_s = src_v[slot, pl.ds(k * 16, 16)]
                idx_d = dst_v[slot, pl.ds(k * 16, 16)]
                raw = (plsc.load_gather(as_v, [idx_s])
                       + plsc.load_gather(ad_v, [idx_d]))
                e = jnp.maximum(raw, 0.2 * raw)
                w_v[pl.ds(k * 16, 16)] = jnp.exp(e - cshift)
            cp.wait()

            def scale(g, _):
                w16 = w_v[pl.ds(g * 16, 16)]
                for lane in range(16):
                    wv = jnp.full((16,), w16[lane], _f32)
                    ei = g * 16 + lane
                    for k in range(D // 16):
                        rows_v[ei, pl.ds(k * 16, 16)] = (
                            rows_v[ei, pl.ds(k * 16, 16)] * wv)
                return 0

            lax.fori_loop(0, EBLK // 16, scale, 0)
            pltpu.sync_copy(rows_v, acc_sh.at[dst_v.at[slot]], add=True)
            pltpu.sync_copy(w_v, den_sh.at[dst_v.at[slot]], add=True)
            jn = jnp.minimum(j + 2, nblk - 1)
            pltpu.async_copy(src_hbm.at[c, s, jn], src_v.at[slot],
                             sem_i.at[slot])
            pltpu.async_copy(dst_hbm.at[c, s, jn], dst_v.at[slot],
                             sem_i.at[slot])
            return 0

        lax.fori_loop(0, nblk, blk, 0)
        for slot in range(2):
            pltpu.make_async_copy(src_hbm.at[c, s, 0], src_v.at[slot],
                                  sem_i.at[slot]).wait()
            pltpu.make_async_copy(dst_hbm.at[c, s, 0], dst_v.at[slot],
                                  sem_i.at[slot]).wait()
        plsc.subcore_barrier()

        for r in range(ROWS_PER_TILE // EBLK):
            sl = pl.ds(base + r * EBLK, EBLK)
            pltpu.sync_copy(acc_sh.at[sl], acc_hbm.at[c, sl])
        pltpu.sync_copy(den_sh.at[pl.ds(base, ROWS_PER_TILE)],
                        den_hbm.at[c, pl.ds(base, ROWS_PER_TILE)])

    return body(h, asv, adv, c16, src_p, dst_p)


def kernel(x, edge_index, W1, a_src1, a_dst1, b1, W2, a_src2, a_dst2, b2):
    n = x.shape[0]
    loops = jnp.arange(n, dtype=jnp.int32)
    src = jnp.concatenate([edge_index[0].astype(jnp.int32), loops])
    dst = jnp.concatenate([edge_index[1].astype(jnp.int32), loops])
    e_total = src.shape[0]
    nblk = -(-e_total // (NW * EBLK))
    e_pad = NW * nblk * EBLK
    sent = jnp.int32(n)
    src_p = jnp.full((e_pad,), sent, jnp.int32).at[:e_total].set(src)
    dst_p = jnp.full((e_pad,), sent, jnp.int32).at[:e_total].set(dst)
    src_p = src_p.reshape(NC, NS, nblk, EBLK)
    dst_p = dst_p.reshape(NC, NS, nblk, EBLK)

    x_pad = jnp.pad(x, ((0, N_PAD - n), (0, 0)))
    h1, as1v, ad1v, c1 = _mm_att(x_pad, W1, a_src1, a_dst1)
    acc1, den1 = _sc_edge_pass(h1, as1v, ad1v, c1, src_p, dst_p, nblk)
    h2, as2v, ad2v, c2 = _comb_mm(acc1, den1, b1, W2, a_src2, a_dst2)
    acc2, den2 = _sc_edge_pass(h2, as2v, ad2v, c2, src_p, dst_p, nblk)
    out = _final(acc2, den2, b2)
    return out[:n]
